# baseline (speedup 1.0000x reference)
import jax
import jax.numpy as jnp
from jax.experimental import pallas as pl


def _bn(x, g, b, eps=1e-5):
    mean = jnp.mean(x, axis=0, keepdims=True)
    var = jnp.var(x, axis=0, keepdims=True)
    return (x - mean) / jnp.sqrt(var + eps) * g + b


def _silu(x):
    return x * jax.nn.sigmoid(x)


def _mlp(x, W, b, g, be):
    return _silu(_bn(x @ W + b, g, be))


def _bottleneck(x, q):
    h = _mlp(x, q["W1"], q["b1"], q["g1"], q["be1"])
    return _mlp(h, q["W2"], q["b2"], q["g2"], q["be2"])


def _expand(x, q):
    return _mlp(x, q["W"], q["b"], q["g"], q["be"])


def _egconv(node_feats, edge_feats, src, dst, n_nodes, q):
    e_src = node_feats @ q["Wsg"]
    e_dst = node_feats @ q["Wdg"]
    m = e_src[src] + e_dst[dst] + edge_feats @ q["Weg"]
    sigma = jax.nn.sigmoid(m)
    Bh = node_feats @ q["Wdu"]
    sum_sigma_h = jax.ops.segment_sum(Bh[src] * sigma, dst, num_segments=n_nodes)
    sum_sigma = jax.ops.segment_sum(sigma, dst, num_segments=n_nodes)
    h = sum_sigma_h / (sum_sigma + 1e-6)
    x_out = node_feats @ q["Wsu"] + h
    x_out = _silu(_bn(x_out, q["gn"], q["bn"]))
    y_out = _silu(_bn(m, q["ge"], q["be"]))
    return x_out, y_out


def _add_kernel(a_ref, b_ref, o_ref):
    o_ref[...] = a_ref[...] + b_ref[...]


def _pallas_add(a, b):
    n, d = a.shape
    blk = 2000
    return pl.pallas_call(
        _add_kernel,
        grid=(n // blk,),
        in_specs=[pl.BlockSpec((blk, d), lambda i: (i, 0)),
                  pl.BlockSpec((blk, d), lambda i: (i, 0))],
        out_specs=pl.BlockSpec((blk, d), lambda i: (i, 0)),
        out_shape=jax.ShapeDtypeStruct((n, d), a.dtype),
    )(a, b)


def kernel(x, y, z, edge_index, lg_edge_index, params):
    x_in, y_in, z_in = x, y, z
    xb = _bottleneck(x, params["node_bn"])
    yb = _bottleneck(y, params["pair_bn"])
    zb = _bottleneck(z, params["trip_bn"])
    src_l, dst_l = lg_edge_index[0], lg_edge_index[1]
    m, z2 = _egconv(yb, zb, src_l, dst_l, yb.shape[0], params["edge_upd"])
    src, dst = edge_index[0], edge_index[1]
    x2, y2 = _egconv(xb, m, src, dst, xb.shape[0], params["node_upd"])
    xo = _pallas_add(_expand(x2, params["node_ex"]), x_in)
    yo = _pallas_add(_expand(y2, params["pair_ex"]), y_in)
    zo = _pallas_add(_expand(z2, params["trip_ex"]), z_in)
    return xo, yo, zo


# speedup vs baseline: 4.0679x; 4.0679x over previous
import functools

import jax
import jax.numpy as jnp
from jax import lax
from jax.experimental import pallas as pl
from jax.experimental.pallas import tpu as pltpu, tpu_sc as plsc

BR = 2000


def _mm(a, b):
    return jnp.dot(a, b, preferred_element_type=jnp.float32)


def _bn_act(v, s, n, g, be):
    mean = s[0:1, :] / n
    var = s[1:2, :] / n - mean * mean
    h = (v - mean) / jnp.sqrt(var + 1e-5) * g + be
    return h * jax.nn.sigmoid(h)


def _acc_stats(s_ref, v):
    @pl.when(pl.program_id(0) == 0)
    def _():
        s_ref[...] = jnp.zeros_like(s_ref)

    s_ref[...] += jnp.concatenate(
        [jnp.sum(v, axis=0)[None, :], jnp.sum(v * v, axis=0)[None, :]], axis=0)


def _linstats(X, W, b):
    n, di = X.shape
    do = W.shape[1]
    b = b.reshape(1, do)

    def kern(x_ref, w_ref, b_ref, y_ref, s_ref):
        v = _mm(x_ref[...], w_ref[...]) + b_ref[...]
        y_ref[...] = v
        _acc_stats(s_ref, v)

    return pl.pallas_call(
        kern, grid=(n // BR,),
        in_specs=[pl.BlockSpec((BR, di), lambda i: (i, 0)),
                  pl.BlockSpec((di, do), lambda i: (0, 0)),
                  pl.BlockSpec((1, do), lambda i: (0, 0))],
        out_specs=[pl.BlockSpec((BR, do), lambda i: (i, 0)),
                   pl.BlockSpec((2, do), lambda i: (0, 0))],
        out_shape=[jax.ShapeDtypeStruct((n, do), jnp.float32),
                   jax.ShapeDtypeStruct((2, do), jnp.float32)],
    )(X, W, b)


def _bnact_linstats(X, s, g, be, W, b):
    n, di = X.shape
    do = W.shape[1]
    g, be, b = g.reshape(1, di), be.reshape(1, di), b.reshape(1, do)

    def kern(x_ref, s_ref, g_ref, be_ref, w_ref, b_ref, y_ref, so_ref):
        h = _bn_act(x_ref[...], s_ref[...], n, g_ref[...], be_ref[...])
        v = _mm(h, w_ref[...]) + b_ref[...]
        y_ref[...] = v
        _acc_stats(so_ref, v)

    return pl.pallas_call(
        kern, grid=(n // BR,),
        in_specs=[pl.BlockSpec((BR, di), lambda i: (i, 0)),
                  pl.BlockSpec((2, di), lambda i: (0, 0)),
                  pl.BlockSpec((1, di), lambda i: (0, 0)),
                  pl.BlockSpec((1, di), lambda i: (0, 0)),
                  pl.BlockSpec((di, do), lambda i: (0, 0)),
                  pl.BlockSpec((1, do), lambda i: (0, 0))],
        out_specs=[pl.BlockSpec((BR, do), lambda i: (i, 0)),
                   pl.BlockSpec((2, do), lambda i: (0, 0))],
        out_shape=[jax.ShapeDtypeStruct((n, do), jnp.float32),
                   jax.ShapeDtypeStruct((2, do), jnp.float32)],
    )(X, s, g, be, W, b)


def _bnact_linmulti(X, s, g, be, Ws):
    n, di = X.shape
    dos = [W.shape[1] for W in Ws]
    g, be = g.reshape(1, di), be.reshape(1, di)

    def kern(*refs):
        x_ref, s_ref, g_ref, be_ref = refs[:4]
        w_refs = refs[4:4 + len(Ws)]
        y_refs = refs[4 + len(Ws):]
        h = _bn_act(x_ref[...], s_ref[...], n, g_ref[...], be_ref[...])
        for w_ref, y_ref in zip(w_refs, y_refs):
            y_ref[...] = _mm(h, w_ref[...])

    return pl.pallas_call(
        kern, grid=(n // BR,),
        in_specs=[pl.BlockSpec((BR, di), lambda i: (i, 0)),
                  pl.BlockSpec((2, di), lambda i: (0, 0)),
                  pl.BlockSpec((1, di), lambda i: (0, 0)),
                  pl.BlockSpec((1, di), lambda i: (0, 0))] +
                 [pl.BlockSpec((di, do), lambda i: (0, 0)) for do in dos],
        out_specs=[pl.BlockSpec((BR, do), lambda i: (i, 0)) for do in dos],
        out_shape=[jax.ShapeDtypeStruct((n, do), jnp.float32) for do in dos],
    )(X, s, g, be, *Ws)


def _bnact_res(X, s, g, be, Xin):
    n, di = X.shape
    g, be = g.reshape(1, di), be.reshape(1, di)

    def kern(x_ref, s_ref, g_ref, be_ref, r_ref, o_ref):
        o_ref[...] = _bn_act(x_ref[...], s_ref[...], n, g_ref[...],
                             be_ref[...]) + r_ref[...]

    return pl.pallas_call(
        kern, grid=(n // BR,),
        in_specs=[pl.BlockSpec((BR, di), lambda i: (i, 0)),
                  pl.BlockSpec((2, di), lambda i: (0, 0)),
                  pl.BlockSpec((1, di), lambda i: (0, 0)),
                  pl.BlockSpec((1, di), lambda i: (0, 0)),
                  pl.BlockSpec((BR, di), lambda i: (i, 0))],
        out_specs=pl.BlockSpec((BR, di), lambda i: (i, 0)),
        out_shape=jax.ShapeDtypeStruct((n, di), jnp.float32),
    )(X, s, g, be, Xin)


def _gate(gA, gB, eg):
    n = gA.shape[0]

    def kern(a_ref, b_ref, e_ref, m_ref, sd_ref, sm_ref):
        a = a_ref[...]
        m = a[:, :32] + b_ref[...] + e_ref[...]
        sig = jax.nn.sigmoid(m)
        m_ref[...] = m
        sd_ref[...] = jnp.concatenate([sig, a[:, 32:] * sig], axis=1)
        _acc_stats(sm_ref, m)

    return pl.pallas_call(
        kern, grid=(n // BR,),
        in_specs=[pl.BlockSpec((BR, 64), lambda i: (i, 0)),
                  pl.BlockSpec((BR, 32), lambda i: (i, 0)),
                  pl.BlockSpec((BR, 32), lambda i: (i, 0))],
        out_specs=[pl.BlockSpec((BR, 32), lambda i: (i, 0)),
                   pl.BlockSpec((BR, 64), lambda i: (i, 0)),
                   pl.BlockSpec((2, 32), lambda i: (0, 0))],
        out_shape=[jax.ShapeDtypeStruct((n, 32), jnp.float32),
                   jax.ShapeDtypeStruct((n, 64), jnp.float32),
                   jax.ShapeDtypeStruct((2, 32), jnp.float32)],
    )(gA, gB, eg)


def _hadd(part, Su):
    n = Su.shape[0]

    def kern(p_ref, su_ref, o_ref, s_ref):
        tot = p_ref[0] + p_ref[1]
        pre = su_ref[...] + tot[:, 32:] / (tot[:, :32] + 1e-6)
        o_ref[...] = pre
        _acc_stats(s_ref, pre)

    return pl.pallas_call(
        kern, grid=(n // BR,),
        in_specs=[pl.BlockSpec((2, BR, 64), lambda i: (0, i, 0)),
                  pl.BlockSpec((BR, 32), lambda i: (i, 0))],
        out_specs=[pl.BlockSpec((BR, 32), lambda i: (i, 0)),
                   pl.BlockSpec((2, 32), lambda i: (0, 0))],
        out_shape=[jax.ShapeDtypeStruct((n, 32), jnp.float32),
                   jax.ShapeDtypeStruct((2, 32), jnp.float32)],
    )(part, Su)


CH = 80


def _sc_gather(table, idx):
    T, Wd = table.shape
    n = idx.shape[0]
    info = plsc.get_sparse_core_info()
    NC, NS = info.num_cores, info.num_subcores
    per = n // (NC * NS)
    nch = per // CH
    mesh = plsc.VectorSubcoreMesh(core_axis_name="c", subcore_axis_name="s")

    @functools.partial(
        pl.kernel, mesh=mesh,
        compiler_params=pltpu.CompilerParams(use_tc_tiling_on_sc=False),
        out_type=jax.ShapeDtypeStruct((n, Wd), jnp.float32),
        scratch_types=[pltpu.VMEM((CH,), jnp.int32),
                       pltpu.VMEM((CH, Wd), jnp.float32),
                       pltpu.SemaphoreType.DMA])
    def k(table_hbm, idx_hbm, out_hbm, idx_v, rows_v, sem):
        wid = lax.axis_index("s") * NC + lax.axis_index("c")
        base0 = wid * per

        def body(c, carry):
            b = base0 + c * CH
            pltpu.sync_copy(idx_hbm.at[pl.ds(b, CH)], idx_v)
            pltpu.async_copy(table_hbm.at[idx_v], rows_v, sem).wait()
            pltpu.sync_copy(rows_v, out_hbm.at[pl.ds(b, CH)])
            return carry

        lax.fori_loop(0, nch, body, 0)

    return k(table, idx)


def _sc_segsum(data, idx, S, P):
    n = data.shape[0]
    info = plsc.get_sparse_core_info()
    NC, NS = info.num_cores, info.num_subcores
    per = n // (NC * NS)
    nch = per // CH
    ACC = S + 16
    zr = ACC // NS
    outS = S // NS
    zeros = jnp.zeros((zr, 64), jnp.float32)
    mesh = plsc.VectorSubcoreMesh(core_axis_name="c", subcore_axis_name="s")

    @functools.partial(
        pl.kernel, mesh=mesh,
        compiler_params=pltpu.CompilerParams(use_tc_tiling_on_sc=False),
        out_type=jax.ShapeDtypeStruct((NC * S * P, 64), jnp.float32),
        scratch_types=[pltpu.VMEM((CH,), jnp.int32),
                       pltpu.VMEM((CH, 64), jnp.float32),
                       pltpu.VMEM_SHARED((ACC, 64), jnp.float32)])
    def k(data_hbm, idx_hbm, z_hbm, out_hbm, idx_v, dat_v, acc):
        cid = lax.axis_index("c")
        sid = lax.axis_index("s")
        wid = sid * NC + cid
        eb0 = wid * per

        def pass_body(p, carry):
            pltpu.sync_copy(z_hbm, acc.at[pl.ds(sid * zr, zr)])
            plsc.subcore_barrier()
            base = p * S

            def chunk(c, cc):
                b = eb0 + c * CH
                pltpu.sync_copy(idx_hbm.at[pl.ds(b, CH)], idx_v)
                for kk in range(CH // 16):
                    v = idx_v[pl.ds(kk * 16, 16)]
                    loc = v - base
                    oob = loc.astype(jnp.uint32) >= jnp.uint32(S)
                    idx_v[pl.ds(kk * 16, 16)] = jnp.where(
                        oob, jnp.int32(S), loc)
                pltpu.sync_copy(data_hbm.at[pl.ds(b, CH)], dat_v)
                pltpu.sync_copy(dat_v, acc.at[idx_v], add=True)
                return cc

            lax.fori_loop(0, nch, chunk, 0)
            plsc.subcore_barrier()
            pltpu.sync_copy(
                acc.at[pl.ds(sid * outS, outS)],
                out_hbm.at[pl.ds(cid * (S * P) + base + sid * outS, outS)])
            plsc.subcore_barrier()
            return carry

        lax.fori_loop(0, P, pass_body, 0)

    return k(data, idx, zeros)


def kernel(x, y, z, edge_index, lg_edge_index, params):
    pn, pp, pt = params["node_bn"], params["pair_bn"], params["trip_bn"]
    eu, nu = params["edge_upd"], params["node_upd"]
    exn, exp_, ext = params["node_ex"], params["pair_ex"], params["trip_ex"]

    def chain(v, q):
        y1, s1 = _linstats(v, q["W1"], q["b1"])
        return _bnact_linstats(y1, s1, q["g1"], q["be1"], q["W2"], q["b2"])

    y2x, s2x = chain(x, pn)
    y2y, s2y = chain(y, pp)
    y2z, s2z = chain(z, pt)

    A_y, B_y, Su_y = _bnact_linmulti(
        y2y, s2y, pp["g2"], pp["be2"],
        [jnp.concatenate([eu["Wsg"], eu["Wdu"]], axis=1), eu["Wdg"], eu["Wsu"]])
    (ezg,) = _bnact_linmulti(y2z, s2z, pt["g2"], pt["be2"], [eu["Weg"]])
    A_x, B_x, Su_x = _bnact_linmulti(
        y2x, s2x, pn["g2"], pn["be2"],
        [jnp.concatenate([nu["Wsg"], nu["Wdu"]], axis=1), nu["Wdg"], nu["Wsu"]])

    src_l, dst_l = lg_edge_index[0], lg_edge_index[1]
    gA = _sc_gather(A_y, src_l)
    gB = _sc_gather(B_y, dst_l)
    m_l, sd_l, sm_l = _gate(gA, gB, ezg)
    part_l = _sc_segsum(sd_l, dst_l, 31248, 11).reshape(2, 31248 * 11, 64)
    pre_y, spre_y = _hadd(part_l, Su_y)
    (eg2,) = _bnact_linmulti(pre_y, spre_y, eu["gn"], eu["bn"], [nu["Weg"]])

    src, dst = edge_index[0], edge_index[1]
    gA2 = _sc_gather(A_x, src)
    gB2 = _sc_gather(B_x, dst)
    m2, sd2, sm2 = _gate(gA2, gB2, eg2)
    part_g = _sc_segsum(sd2, dst, 10000, 1).reshape(2, 10000, 64)
    pre_x, spre_x = _hadd(part_g, Su_x)

    y3x, s3x = _bnact_linstats(pre_x, spre_x, nu["gn"], nu["bn"],
                               exn["W"], exn["b"])
    y3y, s3y = _bnact_linstats(m2, sm2, nu["ge"], nu["be"],
                               exp_["W"], exp_["b"])
    y3z, s3z = _bnact_linstats(m_l, sm_l, eu["ge"], eu["be"],
                               ext["W"], ext["b"])
    xo = _bnact_res(y3x, s3x, exn["g"], exn["be"], x)
    yo = _bnact_res(y3y, s3y, exp_["g"], exp_["be"], y)
    zo = _bnact_res(y3z, s3z, ext["g"], ext["be"], z)
    return xo, yo, zo
